# baseline (speedup 1.0000x reference)
import jax
import jax.numpy as jnp
from jax.experimental import pallas as pl
from jax.experimental.pallas import tpu as pltpu

_TOK_BLOCK = 1024


def _router_kernel(x0_ref, x1_ref, w_ref, b_ref, probs_ref, idx_ref, wts_ref):
    def half(x):
        logits = jax.lax.dot_general(
            x, w_ref[...], (((1,), (1,)), ((), ())),
            preferred_element_type=jnp.float32,
        )
        logits = logits + b_ref[...]
        cols = jax.lax.broadcasted_iota(jnp.int32, logits.shape, 1)
        n = logits.shape[1]
        v1 = jnp.max(logits, axis=1, keepdims=True)
        i1 = jnp.min(jnp.where(logits == v1, cols, n), axis=1, keepdims=True)
        masked = jnp.where(cols == i1, -jnp.inf, logits)
        v2 = jnp.max(masked, axis=1, keepdims=True)
        i2 = jnp.min(jnp.where(masked == v2, cols, n), axis=1, keepdims=True)
        w1 = 1.0 / (1.0 + jnp.exp(v2 - v1))
        e = jnp.exp(logits - v1)
        z = jnp.sum(e, axis=1, keepdims=True)
        return (e / z, jnp.concatenate([i1, i2], axis=1),
                jnp.concatenate([w1, 1.0 - w1], axis=1))

    p0, ix0, wt0 = half(x0_ref[0])
    p1, ix1, wt1 = half(x1_ref[0])
    probs_ref[0] = p0
    probs_ref[1] = p1
    idx_ref[0] = ix0
    idx_ref[1] = ix1
    wts_ref[0] = wt0
    wts_ref[1] = wt1


@jax.jit
def kernel(x, W, b):
    n_tok, d_model = x.shape
    n_exp = W.shape[0]
    t = _TOK_BLOCK
    h = n_tok // 2
    xr = x.reshape(2, h, d_model)
    probs, idx, wts = pl.pallas_call(
        _router_kernel,
        grid=(h // t,),
        in_specs=[
            pl.BlockSpec((1, t, d_model), lambda i: (0, i, 0)),
            pl.BlockSpec((1, t, d_model), lambda i: (1, i, 0)),
            pl.BlockSpec((n_exp, d_model), lambda i: (0, 0)),
            pl.BlockSpec((1, n_exp), lambda i: (0, 0)),
        ],
        out_specs=[
            pl.BlockSpec((2, t, n_exp), lambda i: (0, i, 0)),
            pl.BlockSpec((2, t, 2), lambda i: (0, i, 0)),
            pl.BlockSpec((2, t, 2), lambda i: (0, i, 0)),
        ],
        out_shape=[
            jax.ShapeDtypeStruct((2, h, n_exp), jnp.float32),
            jax.ShapeDtypeStruct((2, h, 2), jnp.int32),
            jax.ShapeDtypeStruct((2, h, 2), jnp.float32),
        ],
        compiler_params=pltpu.CompilerParams(
            dimension_semantics=("parallel",),
        ),
    )(xr, xr, W.reshape(n_exp, d_model), b.reshape(1, n_exp))
    return (probs.reshape(n_tok, n_exp), idx.reshape(n_tok, 2),
            wts.reshape(n_tok, 2))


# speedup vs baseline: 1.0086x; 1.0086x over previous
import jax
import jax.numpy as jnp
from jax.experimental import pallas as pl
from jax.experimental.pallas import tpu as pltpu

_TOK_BLOCK = 1024


def _router_kernel(x0_ref, x1_ref, w_ref, b_ref, probs_ref, idx_ref, wts_ref):
    def half(x):
        lt = jax.lax.dot_general(
            w_ref[...], x, (((1,), (1,)), ((), ())),
            preferred_element_type=jnp.float32,
        )
        lt = lt + b_ref[...]
        rows = jax.lax.broadcasted_iota(jnp.int32, lt.shape, 0)
        n = lt.shape[0]
        v1 = jnp.max(lt, axis=0, keepdims=True)
        i1 = jnp.min(jnp.where(lt == v1, rows, n), axis=0, keepdims=True)
        masked = jnp.where(rows == i1, -jnp.inf, lt)
        v2 = jnp.max(masked, axis=0, keepdims=True)
        i2 = jnp.min(jnp.where(masked == v2, rows, n), axis=0, keepdims=True)
        w1 = 1.0 / (1.0 + jnp.exp(v2 - v1))
        e = jnp.exp(lt - v1)
        z = jnp.sum(e, axis=0, keepdims=True)
        probs = jnp.transpose(e / z)
        idx = jnp.transpose(jnp.concatenate([i1, i2], axis=0))
        wts = jnp.transpose(jnp.concatenate([w1, 1.0 - w1], axis=0))
        return probs, idx, wts

    p0, ix0, wt0 = half(x0_ref[0])
    p1, ix1, wt1 = half(x1_ref[0])
    probs_ref[0] = p0
    probs_ref[1] = p1
    idx_ref[0] = ix0
    idx_ref[1] = ix1
    wts_ref[0] = wt0
    wts_ref[1] = wt1


@jax.jit
def kernel(x, W, b):
    n_tok, d_model = x.shape
    n_exp = W.shape[0]
    t = _TOK_BLOCK
    h = n_tok // 2
    xr = x.reshape(2, h, d_model)
    probs, idx, wts = pl.pallas_call(
        _router_kernel,
        grid=(h // t,),
        in_specs=[
            pl.BlockSpec((1, t, d_model), lambda i: (0, i, 0)),
            pl.BlockSpec((1, t, d_model), lambda i: (1, i, 0)),
            pl.BlockSpec((n_exp, d_model), lambda i: (0, 0)),
            pl.BlockSpec((n_exp, 1), lambda i: (0, 0)),
        ],
        out_specs=[
            pl.BlockSpec((2, t, n_exp), lambda i: (0, i, 0)),
            pl.BlockSpec((2, t, 2), lambda i: (0, i, 0)),
            pl.BlockSpec((2, t, 2), lambda i: (0, i, 0)),
        ],
        out_shape=[
            jax.ShapeDtypeStruct((2, h, n_exp), jnp.float32),
            jax.ShapeDtypeStruct((2, h, 2), jnp.int32),
            jax.ShapeDtypeStruct((2, h, 2), jnp.float32),
        ],
        compiler_params=pltpu.CompilerParams(
            dimension_semantics=("parallel",),
        ),
    )(xr, xr, W.reshape(n_exp, d_model), b.reshape(n_exp, 1))
    return (probs.reshape(n_tok, n_exp), idx.reshape(n_tok, 2),
            wts.reshape(n_tok, 2))


# speedup vs baseline: 1.3086x; 1.2974x over previous
import jax
import jax.numpy as jnp
from jax.experimental import pallas as pl
from jax.experimental.pallas import tpu as pltpu

_TOK_BLOCK = 1024


def _router_kernel(x0_ref, x1_ref, w_ref, b_ref, probs_ref, idx_ref, wts_ref):
    def half(x):
        lt = jax.lax.dot_general(
            w_ref[...], x, (((1,), (1,)), ((), ())),
            preferred_element_type=jnp.float32,
        )
        lt = lt + b_ref[...]
        rows = jax.lax.broadcasted_iota(jnp.int32, lt.shape, 0)
        n = lt.shape[0]
        v1 = jnp.max(lt, axis=0, keepdims=True)
        i1 = jnp.min(jnp.where(lt == v1, rows, n), axis=0, keepdims=True)
        masked = jnp.where(rows == i1, -jnp.inf, lt)
        v2 = jnp.max(masked, axis=0, keepdims=True)
        i2 = jnp.min(jnp.where(masked == v2, rows, n), axis=0, keepdims=True)
        w1 = 1.0 / (1.0 + jnp.exp(v2 - v1))
        e = jnp.exp(lt - v1)
        z = jnp.sum(e, axis=0, keepdims=True)
        probs = jnp.transpose(e / z)
        idx = jnp.concatenate([i1, i2], axis=0)
        wts = jnp.concatenate([w1, 1.0 - w1], axis=0)
        return probs, idx, wts

    p0, ix0, wt0 = half(x0_ref[0])
    p1, ix1, wt1 = half(x1_ref[0])
    probs_ref[0] = p0
    probs_ref[1] = p1
    idx_ref[0] = ix0
    idx_ref[1] = ix1
    wts_ref[0] = wt0
    wts_ref[1] = wt1


@jax.jit
def kernel(x, W, b):
    n_tok, d_model = x.shape
    n_exp = W.shape[0]
    t = _TOK_BLOCK
    h = n_tok // 2
    xr = x.reshape(2, h, d_model)
    probs, idx_t, wts_t = pl.pallas_call(
        _router_kernel,
        grid=(h // t,),
        in_specs=[
            pl.BlockSpec((1, t, d_model), lambda i: (0, i, 0)),
            pl.BlockSpec((1, t, d_model), lambda i: (1, i, 0)),
            pl.BlockSpec((n_exp, d_model), lambda i: (0, 0)),
            pl.BlockSpec((n_exp, 1), lambda i: (0, 0)),
        ],
        out_specs=[
            pl.BlockSpec((2, t, n_exp), lambda i: (0, i, 0)),
            pl.BlockSpec((2, 2, t), lambda i: (0, 0, i)),
            pl.BlockSpec((2, 2, t), lambda i: (0, 0, i)),
        ],
        out_shape=[
            jax.ShapeDtypeStruct((2, h, n_exp), jnp.float32),
            jax.ShapeDtypeStruct((2, 2, h), jnp.int32),
            jax.ShapeDtypeStruct((2, 2, h), jnp.float32),
        ],
        compiler_params=pltpu.CompilerParams(
            dimension_semantics=("parallel",),
        ),
    )(xr, xr, W.reshape(n_exp, d_model), b.reshape(n_exp, 1))
    idx = jnp.transpose(idx_t, (0, 2, 1)).reshape(n_tok, 2)
    wts = jnp.transpose(wts_t, (0, 2, 1)).reshape(n_tok, 2)
    return probs.reshape(n_tok, n_exp), idx, wts
